# baseline (speedup 1.0000x reference)
import functools

import jax
import jax.numpy as jnp
from jax import lax
from jax.experimental import pallas as pl
from jax.experimental.pallas import tpu as pltpu
from jax.experimental.pallas import tpu_sc as plsc

_B = 32
_ROW = 160000

_PERM = (22, 18, 6, 26, 21, 27, 10, 20, 24, 4, 31, 14, 0, 3, 5, 17,
         28, 2, 23, 1, 8, 16, 30, 7, 19, 15, 9, 13, 11, 25, 12, 29)


def _make_remix():
    mesh = plsc.VectorSubcoreMesh(core_axis_name="c", subcore_axis_name="s")

    @functools.partial(
        pl.kernel,
        out_type=jax.ShapeDtypeStruct((2 * _B, 1, _ROW), jnp.float32),
        mesh=mesh,
        scratch_types=[
            pltpu.SemaphoreType.DMA,
            pltpu.SemaphoreType.DMA,
        ],
    )
    def remix(src_hbm, out_hbm, nsem, csem):
        wid = lax.axis_index("s") * 2 + lax.axis_index("c")
        for k in range(_B):
            @pl.when(wid == k)
            def _move(k=k):
                n = pltpu.make_async_copy(
                    src_hbm.at[_PERM[k]], out_hbm.at[k], nsem)
                c = pltpu.make_async_copy(
                    src_hbm.at[_B + k], out_hbm.at[_B + k], csem)
                n.start()
                c.start()
                n.wait()
                c.wait()

    return remix


_remix = _make_remix()


def kernel(sources):
    src = sources.reshape(2 * _B, 1, _ROW)
    out = _remix(src)
    return out.reshape(2, _B, 1, _ROW)


# speedup vs baseline: 26.2065x; 26.2065x over previous
import functools

import jax
import jax.numpy as jnp
from jax import lax
from jax.experimental import pallas as pl
from jax.experimental.pallas import tpu as pltpu
from jax.experimental.pallas import tpu_sc as plsc

_B = 32
_ROW = 160000
_NCH = 10
_CHUNK = _ROW // _NCH
_TOT = 2 * _B * _NCH
_NW = 32
_RPW = _TOT // _NW
_K = 4
_NG = _RPW // _K
_NBUF = 2


def _index_table():
    perm = jnp.argsort(jax.random.uniform(jax.random.key(42), (_B,)))
    idx_noise = (perm[:, None] * _NCH + jnp.arange(_NCH)[None, :]).reshape(-1)
    idx_clean = jnp.arange(_B * _NCH, 2 * _B * _NCH)
    return (
        jnp.concatenate([idx_noise, idx_clean])
        .astype(jnp.int32)
        .reshape(_NW, _NG, _K)
    )


def _make_remix():
    mesh = plsc.VectorSubcoreMesh(core_axis_name="c", subcore_axis_name="s")

    @functools.partial(
        pl.kernel,
        out_type=jax.ShapeDtypeStruct((_TOT, 1, _CHUNK), jnp.float32),
        mesh=mesh,
        scratch_types=(
            [pltpu.VMEM((_NG, _K), jnp.int32)]
            + [pltpu.VMEM((_K, 1, _CHUNK), jnp.float32)] * _NBUF
            + [pltpu.SemaphoreType.DMA] * (2 * _NBUF)
        ),
    )
    def remix(src_hbm, idx_hbm, out_hbm, idx_v, *rest):
        bufs = list(rest[:_NBUF])
        gsem = list(rest[_NBUF:2 * _NBUF])
        ssem = list(rest[2 * _NBUF:])
        wid = lax.axis_index("s") * 2 + lax.axis_index("c")
        base = wid * _RPW
        pltpu.sync_copy(idx_hbm.at[wid], idx_v)
        gh = [None] * _NG
        sh = [None] * _NG
        for g in range(_NG):
            if g >= _NBUF:
                sh[g - _NBUF].wait()
            gh[g] = pltpu.async_copy(
                src_hbm.at[idx_v.at[g]], bufs[g % _NBUF], gsem[g % _NBUF]
            )
            if g >= 1:
                gh[g - 1].wait()
                sh[g - 1] = pltpu.async_copy(
                    bufs[(g - 1) % _NBUF],
                    out_hbm.at[pl.ds(base + (g - 1) * _K, _K)],
                    ssem[(g - 1) % _NBUF],
                )
        gh[_NG - 1].wait()
        sh[_NG - 1] = pltpu.async_copy(
            bufs[(_NG - 1) % _NBUF],
            out_hbm.at[pl.ds(base + (_NG - 1) * _K, _K)],
            ssem[(_NG - 1) % _NBUF],
        )
        for g in range(max(0, _NG - _NBUF), _NG):
            sh[g].wait()

    return remix


_remix = _make_remix()


def kernel(sources):
    src = sources.reshape(_TOT, 1, _CHUNK)
    out = _remix(src, _index_table())
    return out.reshape(2, _B, 1, _ROW)
